# Initial kernel scaffold; baseline (speedup 1.0000x reference)
import jax
import jax.numpy as jnp
from jax.experimental import pallas as pl


def kernel(qs, ras, decs, mag_raw):
    raise NotImplementedError("write your pallas kernel here")


# speedup vs baseline: 27.2951x; 27.2951x over previous
import functools

import jax
import jax.numpy as jnp
import numpy as np
from jax import lax
from jax.experimental import pallas as pl
from jax.experimental.pallas import tpu as pltpu
from jax.experimental.pallas import tpu_sc as plsc

_B = 2
_N = 128
_R = _B * _N
_IMG = 512
_PLANE = _IMG * _IMG
_OUT = _B * _PLANE

_XK = np.arange(-3, 4, dtype=np.float32)
_K1 = np.exp(-(_XK * _XK) / np.float32(18.0)).astype(np.float32)
_K1 = (_K1 / _K1.sum(dtype=np.float32)).astype(np.float32)
_K2 = np.outer(_K1, _K1).astype(np.float32)
_K2B = np.asarray(
    jnp.asarray(_K2).astype(jnp.bfloat16).astype(jnp.float32))
_OFFS = [(dx, dy) for dx in range(-3, 4) for dy in range(-3, 4)]
_NTAP = len(_OFFS)
_TAPS = _NTAP * _R

_NC = 2
_NS = 16
_NW = _NC * _NS
_SLICE = _OUT // _NW
_LANES = 16


def _plateu(v):
    return 1.0 / ((jnp.exp(100.0 * v - 50.0) + 1.0) * (jnp.exp(-100.0 * v - 50.0) + 1.0))


def _tc_star_body(av, bv, cv, dv, ras, decs, mag, idx_out, val_out):
    lane = lax.broadcasted_iota(jnp.int32, (1, _R), 1)
    a = av[...]
    b = bv[...]
    c = cv[...]
    d = dv[...]
    ras_v = ras[...]
    alt = decs[...]
    magnitude = (8.0 - mag[...]) / 10.0

    az = -ras_v * 15.0 / 180.0 * np.pi
    ux = -(jnp.cos(alt) * jnp.sin(2.0 * np.pi - az))
    uy = -(jnp.cos(alt) * jnp.cos(2.0 * np.pi - az))
    uz = jnp.sin(alt)

    r00 = a * a + b * b - c * c - d * d
    r01 = 2.0 * (b * c - a * d)
    r02 = 2.0 * (b * d + a * c)
    r10 = 2.0 * (b * c + a * d)
    r11 = a * a + c * c - b * b - d * d
    r12 = 2.0 * (c * d - a * b)
    r20 = 2.0 * (b * d - a * c)
    r21 = 2.0 * (c * d + a * b)
    r22 = a * a + d * d - b * b - c * c

    def _b16(x):
        return x.astype(jnp.bfloat16).astype(jnp.float32)

    ux16, uy16, uz16 = _b16(ux), _b16(uy), _b16(uz)
    px = _b16(r00) * ux16 + _b16(r01) * uy16 + _b16(r02) * uz16
    py = _b16(r10) * ux16 + _b16(r11) * uy16 + _b16(r12) * uz16
    pz = _b16(r20) * ux16 + _b16(r21) * uy16 + _b16(r22) * uz16

    alp = jnp.arctan2(py, px)
    dlt = jnp.arctan2(pz, jnp.sqrt(px * px + py * py))
    cdlt = jnp.cos(dlt)
    cs = cdlt * jnp.cos(alp)
    xs = cdlt * jnp.sin(alp)
    ys = jnp.sin(dlt)
    filt = _plateu(xs) * _plateu(ys) * jnp.maximum(cs, 0.0)

    ix = (256.0 + 256.0 * (jnp.fmod(xs, 0.5) / 0.5)).astype(jnp.int32)
    iy = (256.0 + 256.0 * (jnp.fmod(ys, 0.5) / 0.5)).astype(jnp.int32)
    ix = jnp.clip(ix, 0, _IMG - 1)
    iy = jnp.clip(iy, 0, _IMG - 1)
    val = filt * magnitude

    pix = ix * _IMG + iy
    pf = pix.astype(jnp.float32)
    i0 = lax.broadcasted_iota(jnp.int32, (_R, _R), 0)
    i1 = lax.broadcasted_iota(jnp.int32, (_R, _R), 1)
    eye = (i0 == i1).astype(jnp.float32)
    pcol = lax.dot_general(eye, pf, (((1,), (1,)), ((), ())),
                           preferred_element_type=jnp.float32)
    clash = (pcol == pf) & (i0 > i1)
    collide = jnp.any(clash, axis=0, keepdims=True)
    val = jnp.where(collide, 0.0, val)
    val16 = _b16(val)

    bid = (lane >= _N).astype(jnp.int32)
    rows_i = []
    rows_v = []
    for dx, dy in _OFFS:
        xx = ix + dx
        yy = iy + dy
        valid = (xx >= 0) & (xx < _IMG) & (yy >= 0) & (yy < _IMG)
        g = bid * _PLANE + xx * _IMG + yy
        rows_i.append(jnp.where(valid, g, 0))
        rows_v.append(jnp.where(valid, val16 * np.float32(_K2B[dx + 3, dy + 3]), 0.0))
    idx_out[...] = jnp.concatenate(rows_i, axis=0)
    val_out[...] = jnp.concatenate(rows_v, axis=0)


_tc_star = pl.pallas_call(
    _tc_star_body,
    out_shape=(
        jax.ShapeDtypeStruct((_NTAP, _R), jnp.int32),
        jax.ShapeDtypeStruct((_NTAP, _R), jnp.float32),
    ),
)


def _sc_scatter_body(tapidx_hbm, tapval_hbm, out_hbm, idx_v, val_v, acc):
    wid = lax.axis_index("s") * _NC + lax.axis_index("c")
    lo = wid * _SLICE
    pltpu.sync_copy(tapidx_hbm, idx_v)
    pltpu.sync_copy(tapval_hbm, val_v)

    def zero_body(i, carry):
        acc[pl.ds(i * _LANES, _LANES)] = jnp.zeros((_LANES,), jnp.float32)
        return carry

    lax.fori_loop(0, _SLICE // _LANES, zero_body, 0)

    def tap_body(i, carry):
        g = idx_v[pl.ds(i * _LANES, _LANES)]
        v = val_v[pl.ds(i * _LANES, _LANES)]
        loc = g - lo
        m = (loc >= 0) & (loc < _SLICE)
        locc = jnp.clip(loc, 0, _SLICE - 1)
        plsc.addupdate_scatter(acc, [locc], v, mask=m)
        return carry

    lax.fori_loop(0, _TAPS // _LANES, tap_body, 0)
    pltpu.sync_copy(acc, out_hbm.at[pl.ds(lo, _SLICE)])


@functools.cache
def _sc_scatter_kernel():
    return pl.kernel(
        _sc_scatter_body,
        out_type=jax.ShapeDtypeStruct((_OUT,), jnp.float32),
        mesh=plsc.VectorSubcoreMesh(core_axis_name="c", subcore_axis_name="s",
                                    num_cores=_NC, num_subcores=_NS),
        compiler_params=pltpu.CompilerParams(needs_layout_passes=False),
        scratch_types=[
            pltpu.VMEM((_TAPS,), jnp.int32),
            pltpu.VMEM((_TAPS,), jnp.float32),
            pltpu.VMEM((_SLICE,), jnp.float32),
        ],
    )


def kernel(qs, ras, decs, mag_raw):
    av = jnp.broadcast_to(qs[:, 0:1], (_B, _N)).reshape(1, _R)
    bv = jnp.broadcast_to(qs[:, 1:2], (_B, _N)).reshape(1, _R)
    cv = jnp.broadcast_to(qs[:, 2:3], (_B, _N)).reshape(1, _R)
    dv = jnp.broadcast_to(qs[:, 3:4], (_B, _N)).reshape(1, _R)
    ras_rep = jnp.broadcast_to(ras.reshape(_N, 1), (_N, 2)).reshape(1, _R)
    decs_rep = jnp.broadcast_to(decs.reshape(_N, 1), (_N, 2)).reshape(1, _R)
    mag_rep = jnp.broadcast_to(mag_raw.reshape(1, _N), (_B, _N)).reshape(1, _R)

    tap_idx, tap_val = _tc_star(av, bv, cv, dv, ras_rep, decs_rep, mag_rep)
    sky = _sc_scatter_kernel()(tap_idx.reshape(_TAPS), tap_val.reshape(_TAPS))
    return sky.reshape(_B, 1, _IMG, _IMG)


# speedup vs baseline: 38.6654x; 1.4166x over previous
import functools

import jax
import jax.numpy as jnp
import ml_dtypes
import numpy as np
from jax import lax
from jax.experimental import pallas as pl
from jax.experimental.pallas import tpu as pltpu
from jax.experimental.pallas import tpu_sc as plsc

_B = 2
_N = 128
_M = 128
_IMG = 512
_PLANE = _IMG * _IMG
_OUT = _B * _PLANE

_XK = np.arange(-3, 4, dtype=np.float32)
_K1 = np.exp(-(_XK * _XK) / np.float32(18.0)).astype(np.float32)
_K1 = (_K1 / _K1.sum(dtype=np.float32)).astype(np.float32)
_K2 = np.outer(_K1, _K1).astype(np.float32)
_K2B = _K2.astype(ml_dtypes.bfloat16).astype(np.float32)
_OFFS = [(dx, dy) for dx in range(-3, 4) for dy in range(-3, 4)]
_NTAP = len(_OFFS)
_TAPS = _NTAP * _M

_NC = 2
_NS = 16
_NW = _NC * _NS
_SLICE = _OUT // _NW
_LANES = 16


def _plateu(v):
    return 1.0 / ((jnp.exp(100.0 * v - 50.0) + 1.0) * (jnp.exp(-100.0 * v - 50.0) + 1.0))


def _tc_star_body(av, bv, cv, dv, ras, decs, mag, idx_out, val_out):
    lane = lax.broadcasted_iota(jnp.int32, (1, _M), 1)
    a = av[...]
    b = bv[...]
    c = cv[...]
    d = dv[...]
    ras_v = ras[...]
    alt = decs[...]
    magnitude = (8.0 - mag[...]) / 10.0

    az = -ras_v * 15.0 / 180.0 * np.pi
    ux = -(jnp.cos(alt) * jnp.sin(2.0 * np.pi - az))
    uy = -(jnp.cos(alt) * jnp.cos(2.0 * np.pi - az))
    uz = jnp.sin(alt)

    r00 = a * a + b * b - c * c - d * d
    r01 = 2.0 * (b * c - a * d)
    r02 = 2.0 * (b * d + a * c)
    r10 = 2.0 * (b * c + a * d)
    r11 = a * a + c * c - b * b - d * d
    r12 = 2.0 * (c * d - a * b)
    r20 = 2.0 * (b * d - a * c)
    r21 = 2.0 * (c * d + a * b)
    r22 = a * a + d * d - b * b - c * c

    def _b16(x):
        return x.astype(jnp.bfloat16).astype(jnp.float32)

    ux16, uy16, uz16 = _b16(ux), _b16(uy), _b16(uz)
    px = _b16(r00) * ux16 + _b16(r01) * uy16 + _b16(r02) * uz16
    py = _b16(r10) * ux16 + _b16(r11) * uy16 + _b16(r12) * uz16
    pz = _b16(r20) * ux16 + _b16(r21) * uy16 + _b16(r22) * uz16

    alp = jnp.arctan2(py, px)
    dlt = jnp.arctan2(pz, jnp.sqrt(px * px + py * py))
    cdlt = jnp.cos(dlt)
    cs = cdlt * jnp.cos(alp)
    xs = cdlt * jnp.sin(alp)
    ys = jnp.sin(dlt)
    filt = _plateu(xs) * _plateu(ys) * jnp.maximum(cs, 0.0)

    ix = (256.0 + 256.0 * (jnp.fmod(xs, 0.5) / 0.5)).astype(jnp.int32)
    iy = (256.0 + 256.0 * (jnp.fmod(ys, 0.5) / 0.5)).astype(jnp.int32)
    ix = jnp.clip(ix, 0, _IMG - 1)
    iy = jnp.clip(iy, 0, _IMG - 1)
    val = filt * magnitude

    pix = ix * _IMG + iy
    pf = pix.astype(jnp.float32)
    i0 = lax.broadcasted_iota(jnp.int32, (_M, _M), 0)
    i1 = lax.broadcasted_iota(jnp.int32, (_M, _M), 1)
    eye = (i0 == i1).astype(jnp.float32)
    pcol = lax.dot_general(eye, pf, (((1,), (1,)), ((), ())),
                           preferred_element_type=jnp.float32)
    clash = (pcol == pf) & (i0 > i1)
    collide = jnp.any(clash, axis=0, keepdims=True)
    val = jnp.where(collide, 0.0, val)
    val16 = _b16(val)

    bid = (lane >= _M // _B).astype(jnp.int32)
    rows_i = []
    rows_v = []
    for dx, dy in _OFFS:
        xx = ix + dx
        yy = iy + dy
        valid = (xx >= 0) & (xx < _IMG) & (yy >= 0) & (yy < _IMG)
        g = bid * _PLANE + xx * _IMG + yy
        rows_i.append(jnp.where(valid, g, 0))
        rows_v.append(jnp.where(valid, val16 * np.float32(_K2B[dx + 3, dy + 3]), 0.0))
    idx_out[...] = jnp.concatenate(rows_i, axis=0)
    val_out[...] = jnp.concatenate(rows_v, axis=0)


_tc_star = pl.pallas_call(
    _tc_star_body,
    out_shape=(
        jax.ShapeDtypeStruct((_NTAP, _M), jnp.int32),
        jax.ShapeDtypeStruct((_NTAP, _M), jnp.float32),
    ),
)


def _sc_scatter_body(tapidx_hbm, tapval_hbm, out_hbm, idx_v, val_v, acc,
                     sem_i, sem_v):
    wid = lax.axis_index("s") * _NC + lax.axis_index("c")
    lo = wid * _SLICE
    cp_i = pltpu.async_copy(tapidx_hbm, idx_v, sem_i)
    cp_v = pltpu.async_copy(tapval_hbm, val_v, sem_v)

    def zero_body(i, carry):
        acc[pl.ds(i * _LANES, _LANES)] = jnp.zeros((_LANES,), jnp.float32)
        return carry

    lax.fori_loop(0, _SLICE // _LANES, zero_body, 0, unroll=8)
    cp_i.wait()
    cp_v.wait()

    def tap_body(i, carry):
        g = idx_v[pl.ds(i * _LANES, _LANES)]
        v = val_v[pl.ds(i * _LANES, _LANES)]
        loc = g - lo
        m = (loc >= 0) & (loc < _SLICE)
        locc = jnp.clip(loc, 0, _SLICE - 1)
        plsc.addupdate_scatter(acc, [locc], v, mask=m)
        return carry

    lax.fori_loop(0, _TAPS // _LANES, tap_body, 0, unroll=8)
    pltpu.sync_copy(acc, out_hbm.at[pl.ds(lo, _SLICE)])


@functools.cache
def _sc_scatter_kernel():
    return pl.kernel(
        _sc_scatter_body,
        out_type=jax.ShapeDtypeStruct((_OUT,), jnp.float32),
        mesh=plsc.VectorSubcoreMesh(core_axis_name="c", subcore_axis_name="s",
                                    num_cores=_NC, num_subcores=_NS),
        compiler_params=pltpu.CompilerParams(needs_layout_passes=False),
        scratch_types=[
            pltpu.VMEM((_TAPS,), jnp.int32),
            pltpu.VMEM((_TAPS,), jnp.float32),
            pltpu.VMEM((_SLICE,), jnp.float32),
            pltpu.SemaphoreType.DMA,
            pltpu.SemaphoreType.DMA,
        ],
    )


def kernel(qs, ras, decs, mag_raw):
    av = jnp.broadcast_to(qs[:, 0:1], (_B, _M // _B)).reshape(1, _M)
    bv = jnp.broadcast_to(qs[:, 1:2], (_B, _M // _B)).reshape(1, _M)
    cv = jnp.broadcast_to(qs[:, 2:3], (_B, _M // _B)).reshape(1, _M)
    dv = jnp.broadcast_to(qs[:, 3:4], (_B, _M // _B)).reshape(1, _M)
    ras_in = ras.reshape(1, _M)
    decs_in = decs.reshape(1, _M)
    mag_odd = mag_raw.reshape(_N // 2, 2)[:, 1].reshape(1, _N // 2)
    mag_in = jnp.broadcast_to(mag_odd, (_B, _N // 2)).reshape(1, _M)

    tap_idx, tap_val = _tc_star(av, bv, cv, dv, ras_in, decs_in, mag_in)
    sky = _sc_scatter_kernel()(tap_idx.reshape(_TAPS), tap_val.reshape(_TAPS))
    return sky.reshape(_B, 1, _IMG, _IMG)


# speedup vs baseline: 44.0279x; 1.1387x over previous
import functools

import jax
import jax.numpy as jnp
import ml_dtypes
import numpy as np
from jax import lax
from jax.experimental import pallas as pl
from jax.experimental.pallas import tpu as pltpu
from jax.experimental.pallas import tpu_sc as plsc

_B = 2
_N = 128
_M = 128
_IMG = 512
_PLANE = _IMG * _IMG
_OUT = _B * _PLANE

_XK = np.arange(-3, 4, dtype=np.float32)
_K1 = np.exp(-(_XK * _XK) / np.float32(18.0)).astype(np.float32)
_K1 = (_K1 / _K1.sum(dtype=np.float32)).astype(np.float32)
_K2 = np.outer(_K1, _K1).astype(np.float32)
_K2B = _K2.astype(ml_dtypes.bfloat16).astype(np.float32)
_OFFS = [(dx, dy) for dx in range(-3, 4) for dy in range(-3, 4)]
_NTAP = len(_OFFS)
_TAPS = _NTAP * _M
_VIS_EPS = 1e-12

_NC = 2
_NS = 16
_NW = _NC * _NS
_SLICE = _OUT // _NW
_LANES = 16

_HI = lax.Precision.HIGHEST


def _plateu(v):
    return 1.0 / ((jnp.exp(100.0 * v - 50.0) + 1.0) * (jnp.exp(-100.0 * v - 50.0) + 1.0))


def _tc_star_body(qs, ras, decs, mag, idx_out, val_out, cnt_out):
    lane = lax.broadcasted_iota(jnp.int32, (1, _M), 1)
    i0 = lax.broadcasted_iota(jnp.int32, (_M, _M), 0)
    i1 = lax.broadcasted_iota(jnp.int32, (_M, _M), 1)
    eye_b = i0 == i1
    eye_f = eye_b.astype(jnp.float32)

    def transpose_row(row):
        return jnp.sum(jnp.where(eye_b, row, 0.0), axis=1, keepdims=True)

    def transpose_col(col):
        return jnp.sum(jnp.where(eye_b, col, 0.0), axis=0, keepdims=True)

    low = lane < (_M // _B)

    def qsel(j):
        return jnp.where(low,
                         jnp.broadcast_to(qs[0:1, j:j + 1], (1, _M)),
                         jnp.broadcast_to(qs[1:2, j:j + 1], (1, _M)))

    a, b, c, d = qsel(0), qsel(1), qsel(2), qsel(3)
    ras_v = transpose_col(ras[...])
    alt = transpose_col(decs[...])
    modd = 2 * i1 + 1 - _N * (i1 >= (_M // _B)).astype(jnp.int32)
    psel = i0 == modd
    magr = jnp.sum(jnp.where(psel, transpose_row(mag[...]), 0.0),
                   axis=0, keepdims=True)
    magnitude = (8.0 - magr) / 10.0

    az = -ras_v * 15.0 / 180.0 * np.pi
    ux = -(jnp.cos(alt) * jnp.sin(2.0 * np.pi - az))
    uy = -(jnp.cos(alt) * jnp.cos(2.0 * np.pi - az))
    uz = jnp.sin(alt)

    r00 = a * a + b * b - c * c - d * d
    r01 = 2.0 * (b * c - a * d)
    r02 = 2.0 * (b * d + a * c)
    r10 = 2.0 * (b * c + a * d)
    r11 = a * a + c * c - b * b - d * d
    r12 = 2.0 * (c * d - a * b)
    r20 = 2.0 * (b * d - a * c)
    r21 = 2.0 * (c * d + a * b)
    r22 = a * a + d * d - b * b - c * c

    def _b16(x):
        return x.astype(jnp.bfloat16).astype(jnp.float32)

    ux16, uy16, uz16 = _b16(ux), _b16(uy), _b16(uz)
    px = _b16(r00) * ux16 + _b16(r01) * uy16 + _b16(r02) * uz16
    py = _b16(r10) * ux16 + _b16(r11) * uy16 + _b16(r12) * uz16
    pz = _b16(r20) * ux16 + _b16(r21) * uy16 + _b16(r22) * uz16

    alp = jnp.arctan2(py, px)
    dlt = jnp.arctan2(pz, jnp.sqrt(px * px + py * py))
    cdlt = jnp.cos(dlt)
    cs = cdlt * jnp.cos(alp)
    xs = cdlt * jnp.sin(alp)
    ys = jnp.sin(dlt)
    filt = _plateu(xs) * _plateu(ys) * jnp.maximum(cs, 0.0)

    ix = (256.0 + 256.0 * (jnp.fmod(xs, 0.5) / 0.5)).astype(jnp.int32)
    iy = (256.0 + 256.0 * (jnp.fmod(ys, 0.5) / 0.5)).astype(jnp.int32)
    ix = jnp.clip(ix, 0, _IMG - 1)
    iy = jnp.clip(iy, 0, _IMG - 1)
    val = filt * magnitude

    pix = ix * _IMG + iy
    pf = pix.astype(jnp.float32)
    pcol = transpose_row(pf)
    clash = (pcol == pf) & (i0 > i1)
    collide = jnp.any(clash, axis=0, keepdims=True)
    val = jnp.where(collide, 0.0, val)
    val16 = _b16(val)

    vis = val16 > _VIS_EPS
    visf = vis.astype(jnp.float32)
    vcol = transpose_row(visf)
    rank = jnp.sum(jnp.where(i0 < i1, vcol, 0.0), axis=0, keepdims=True)
    rank_col = transpose_row(rank)
    sel = ((rank_col == i1.astype(jnp.float32)) & (vcol > 0.5))
    self_f = sel.astype(jnp.float32)
    cnt_out[...] = jnp.sum(visf, axis=1, keepdims=True).astype(
        jnp.int32) * jnp.ones((1, _M), jnp.int32)

    bid = (lane >= _M // _B).astype(jnp.int32)
    rows_i = []
    rows_v = []
    for dx, dy in _OFFS:
        xx = ix + dx
        yy = iy + dy
        valid = (xx >= 0) & (xx < _IMG) & (yy >= 0) & (yy < _IMG)
        g = bid * _PLANE + xx * _IMG + yy
        rows_i.append(jnp.where(valid, g, 0))
        rows_v.append(jnp.where(valid, val16 * np.float32(_K2B[dx + 3, dy + 3]), 0.0))
    gmat = jnp.concatenate(rows_i, axis=0).astype(jnp.float32)
    vmat = jnp.concatenate(rows_v, axis=0)
    idx_out[...] = lax.dot_general(gmat, self_f, (((1,), (0,)), ((), ())),
                                   precision=_HI,
                                   preferred_element_type=jnp.float32
                                   ).astype(jnp.int32)
    val_out[...] = lax.dot_general(vmat, self_f, (((1,), (0,)), ((), ())),
                                   precision=_HI,
                                   preferred_element_type=jnp.float32)


_tc_star = pl.pallas_call(
    _tc_star_body,
    out_shape=(
        jax.ShapeDtypeStruct((_NTAP, _M), jnp.int32),
        jax.ShapeDtypeStruct((_NTAP, _M), jnp.float32),
        jax.ShapeDtypeStruct((1, _M), jnp.int32),
    ),
)


def _sc_scatter_body(tapidx_hbm, tapval_hbm, cnt_hbm, out_hbm,
                     idx_v, val_v, cnt_v, acc, sem_i, sem_v):
    wid = lax.axis_index("s") * _NC + lax.axis_index("c")
    lo = wid * _SLICE
    cp_i = pltpu.async_copy(tapidx_hbm, idx_v, sem_i)
    cp_v = pltpu.async_copy(tapval_hbm, val_v, sem_v)
    pltpu.sync_copy(cnt_hbm.at[pl.ds(0, _LANES)], cnt_v)
    nvis = jnp.max(cnt_v[...])
    nvreg = (nvis + _LANES - 1) // _LANES

    def zero_body(i, carry):
        acc[pl.ds(i * _LANES, _LANES)] = jnp.zeros((_LANES,), jnp.float32)
        return carry

    lax.fori_loop(0, _SLICE // _LANES, zero_body, 0, unroll=8)
    cp_i.wait()
    cp_v.wait()

    def make_tap_body(t):
        def tap_body(j, carry):
            g = idx_v[pl.ds(t * _M + j * _LANES, _LANES)]
            v = val_v[pl.ds(t * _M + j * _LANES, _LANES)]
            loc = g - lo
            m = (loc >= 0) & (loc < _SLICE)
            locc = jnp.clip(loc, 0, _SLICE - 1)
            plsc.addupdate_scatter(acc, [locc], v, mask=m)
            return carry
        return tap_body

    for t in range(_NTAP):
        lax.fori_loop(0, nvreg, make_tap_body(t), 0)

    pltpu.sync_copy(acc, out_hbm.at[pl.ds(lo, _SLICE)])


@functools.cache
def _sc_scatter_kernel():
    return pl.kernel(
        _sc_scatter_body,
        out_type=jax.ShapeDtypeStruct((_OUT,), jnp.float32),
        mesh=plsc.VectorSubcoreMesh(core_axis_name="c", subcore_axis_name="s",
                                    num_cores=_NC, num_subcores=_NS),
        compiler_params=pltpu.CompilerParams(needs_layout_passes=False),
        scratch_types=[
            pltpu.VMEM((_TAPS,), jnp.int32),
            pltpu.VMEM((_TAPS,), jnp.float32),
            pltpu.VMEM((_LANES,), jnp.int32),
            pltpu.VMEM((_SLICE,), jnp.float32),
            pltpu.SemaphoreType.DMA,
            pltpu.SemaphoreType.DMA,
        ],
    )


def kernel(qs, ras, decs, mag_raw):
    tap_idx, tap_val, cnt = _tc_star(qs, ras, decs, mag_raw)
    sky = _sc_scatter_kernel()(tap_idx.reshape(_TAPS), tap_val.reshape(_TAPS),
                               cnt.reshape(_M))
    return sky.reshape(_B, 1, _IMG, _IMG)


# speedup vs baseline: 49.6322x; 1.1273x over previous
import functools

import jax
import jax.numpy as jnp
import ml_dtypes
import numpy as np
from jax import lax
from jax.experimental import pallas as pl
from jax.experimental.pallas import tpu as pltpu
from jax.experimental.pallas import tpu_sc as plsc

_B = 2
_N = 128
_M = 128
_IMG = 512
_PLANE = _IMG * _IMG
_OUT = _B * _PLANE

_XK = np.arange(-3, 4, dtype=np.float32)
_K1 = np.exp(-(_XK * _XK) / np.float32(18.0)).astype(np.float32)
_K1 = (_K1 / _K1.sum(dtype=np.float32)).astype(np.float32)
_K2 = np.outer(_K1, _K1).astype(np.float32)
_K2B = _K2.astype(ml_dtypes.bfloat16).astype(np.float32)
_OFFS = [(dx, dy) for dx in range(-3, 4) for dy in range(-3, 4)]
_NTAP = len(_OFFS)
_VIS_EPS = 1e-12

_NC = 2
_NS = 16
_NW = _NC * _NS
_SLICE = _OUT // _NW
_LANES = 16

_HI = lax.Precision.HIGHEST


def _plateu(v):
    return 1.0 / ((jnp.exp(100.0 * v - 50.0) + 1.0) * (jnp.exp(-100.0 * v - 50.0) + 1.0))


def _tc_star_body(qs, ras, decs, mag, pix_out, val_out, cnt_out):
    lane = lax.broadcasted_iota(jnp.int32, (1, _M), 1)
    i0 = lax.broadcasted_iota(jnp.int32, (_M, _M), 0)
    i1 = lax.broadcasted_iota(jnp.int32, (_M, _M), 1)
    eye_b = i0 == i1

    def transpose_row(row):
        return jnp.sum(jnp.where(eye_b, row, 0.0), axis=1, keepdims=True)

    def transpose_col(col):
        return jnp.sum(jnp.where(eye_b, col, 0.0), axis=0, keepdims=True)

    low = lane < (_M // _B)

    def qsel(j):
        return jnp.where(low,
                         jnp.broadcast_to(qs[0:1, j:j + 1], (1, _M)),
                         jnp.broadcast_to(qs[1:2, j:j + 1], (1, _M)))

    a, b, c, d = qsel(0), qsel(1), qsel(2), qsel(3)
    ras_v = transpose_col(ras[...])
    alt = transpose_col(decs[...])
    modd = 2 * i1 + 1 - _N * (i1 >= (_M // _B)).astype(jnp.int32)
    psel = i0 == modd
    magr = jnp.sum(jnp.where(psel, transpose_row(mag[...]), 0.0),
                   axis=0, keepdims=True)
    magnitude = (8.0 - magr) / 10.0

    az = -ras_v * 15.0 / 180.0 * np.pi
    ux = -(jnp.cos(alt) * jnp.sin(2.0 * np.pi - az))
    uy = -(jnp.cos(alt) * jnp.cos(2.0 * np.pi - az))
    uz = jnp.sin(alt)

    r00 = a * a + b * b - c * c - d * d
    r01 = 2.0 * (b * c - a * d)
    r02 = 2.0 * (b * d + a * c)
    r10 = 2.0 * (b * c + a * d)
    r11 = a * a + c * c - b * b - d * d
    r12 = 2.0 * (c * d - a * b)
    r20 = 2.0 * (b * d - a * c)
    r21 = 2.0 * (c * d + a * b)
    r22 = a * a + d * d - b * b - c * c

    def _b16(x):
        return x.astype(jnp.bfloat16).astype(jnp.float32)

    ux16, uy16, uz16 = _b16(ux), _b16(uy), _b16(uz)
    px = _b16(r00) * ux16 + _b16(r01) * uy16 + _b16(r02) * uz16
    py = _b16(r10) * ux16 + _b16(r11) * uy16 + _b16(r12) * uz16
    pz = _b16(r20) * ux16 + _b16(r21) * uy16 + _b16(r22) * uz16

    alp = jnp.arctan2(py, px)
    dlt = jnp.arctan2(pz, jnp.sqrt(px * px + py * py))
    cdlt = jnp.cos(dlt)
    cs = cdlt * jnp.cos(alp)
    xs = cdlt * jnp.sin(alp)
    ys = jnp.sin(dlt)
    filt = _plateu(xs) * _plateu(ys) * jnp.maximum(cs, 0.0)

    ix = (256.0 + 256.0 * (jnp.fmod(xs, 0.5) / 0.5)).astype(jnp.int32)
    iy = (256.0 + 256.0 * (jnp.fmod(ys, 0.5) / 0.5)).astype(jnp.int32)
    ix = jnp.clip(ix, 0, _IMG - 1)
    iy = jnp.clip(iy, 0, _IMG - 1)
    val = filt * magnitude

    pix = ix * _IMG + iy
    pf = pix.astype(jnp.float32)
    pcol = transpose_row(pf)
    clash = (pcol == pf) & (i0 > i1)
    collide = jnp.any(clash, axis=0, keepdims=True)
    val = jnp.where(collide, 0.0, val)
    val16 = _b16(val)

    vis = val16 > _VIS_EPS
    visf = vis.astype(jnp.float32)
    vcol = transpose_row(visf)
    rank = jnp.sum(jnp.where(i0 < i1, vcol, 0.0), axis=0, keepdims=True)
    rank_col = transpose_row(rank)
    sel_f = ((rank_col == i1.astype(jnp.float32)) &
             (vcol > 0.5)).astype(jnp.float32)
    cnt_out[...] = jnp.sum(visf, axis=1, keepdims=True).astype(
        jnp.int32) * jnp.ones((1, _M), jnp.int32)

    bid = (lane >= _M // _B).astype(jnp.int32)
    gbase = (bid * _PLANE + pix).astype(jnp.float32)
    pix_out[...] = lax.dot_general(gbase, sel_f, (((1,), (0,)), ((), ())),
                                   precision=_HI,
                                   preferred_element_type=jnp.float32
                                   ).astype(jnp.int32)
    val_out[...] = lax.dot_general(val16, sel_f, (((1,), (0,)), ((), ())),
                                   precision=_HI,
                                   preferred_element_type=jnp.float32)


_tc_star = pl.pallas_call(
    _tc_star_body,
    out_shape=(
        jax.ShapeDtypeStruct((1, _M), jnp.int32),
        jax.ShapeDtypeStruct((1, _M), jnp.float32),
        jax.ShapeDtypeStruct((1, _M), jnp.int32),
    ),
)


def _sc_scatter_body(pix_hbm, val_hbm, cnt_hbm, out_hbm,
                     pix_v, val_v, cnt_v, acc, sem_i, sem_v):
    wid = lax.axis_index("s") * _NC + lax.axis_index("c")
    lo = wid * _SLICE
    cp_i = pltpu.async_copy(pix_hbm, pix_v, sem_i)
    cp_v = pltpu.async_copy(val_hbm, val_v, sem_v)
    pltpu.sync_copy(cnt_hbm.at[pl.ds(0, _LANES)], cnt_v)
    nvis = jnp.max(cnt_v[...])
    nvreg = (nvis + _LANES - 1) // _LANES

    def zero_body(i, carry):
        acc[pl.ds(i * _LANES, _LANES)] = jnp.zeros((_LANES,), jnp.float32)
        return carry

    lax.fori_loop(0, _SLICE // _LANES, zero_body, 0, unroll=8)
    cp_i.wait()
    cp_v.wait()

    def star_body(j, carry):
        gb = pix_v[pl.ds(j * _LANES, _LANES)]
        v = val_v[pl.ds(j * _LANES, _LANES)]
        ixl = (gb & (_PLANE - 1)) >> 9
        iyl = gb & (_IMG - 1)
        for dx in range(-3, 4):
            xx = ixl + dx
            vx = (xx >= 0) & (xx < _IMG)
            gx = gb + dx * _IMG - lo
            for dy in range(-3, 4):
                yy = iyl + dy
                m = vx & (yy >= 0) & (yy < _IMG)
                loc = gx + dy
                m = m & (loc >= 0) & (loc < _SLICE)
                locc = jnp.clip(loc, 0, _SLICE - 1)
                tv = v * np.float32(_K2B[dx + 3, dy + 3])
                plsc.addupdate_scatter(acc, [locc], tv, mask=m)
        return carry

    lax.fori_loop(0, nvreg, star_body, 0)
    pltpu.sync_copy(acc, out_hbm.at[pl.ds(lo, _SLICE)])


@functools.cache
def _sc_scatter_kernel():
    return pl.kernel(
        _sc_scatter_body,
        out_type=jax.ShapeDtypeStruct((_OUT,), jnp.float32),
        mesh=plsc.VectorSubcoreMesh(core_axis_name="c", subcore_axis_name="s",
                                    num_cores=_NC, num_subcores=_NS),
        compiler_params=pltpu.CompilerParams(needs_layout_passes=False),
        scratch_types=[
            pltpu.VMEM((_M,), jnp.int32),
            pltpu.VMEM((_M,), jnp.float32),
            pltpu.VMEM((_LANES,), jnp.int32),
            pltpu.VMEM((_SLICE,), jnp.float32),
            pltpu.SemaphoreType.DMA,
            pltpu.SemaphoreType.DMA,
        ],
    )


def kernel(qs, ras, decs, mag_raw):
    pix_c, val_c, cnt = _tc_star(qs, ras, decs, mag_raw)
    sky = _sc_scatter_kernel()(pix_c.reshape(_M), val_c.reshape(_M),
                               cnt.reshape(_M))
    return sky.reshape(_B, 1, _IMG, _IMG)
